# baseline (speedup 1.0000x reference)
import functools

import jax
import jax.numpy as jnp
from jax.experimental import pallas as pl
from jax.experimental.pallas import tpu as pltpu

FEATS = 16
LANES = 128
COL_BLOCK = 32768


def _mlp_t_body(xt_ref, w1_ref, b1_ref, w2_ref, b2_ref, o_ref):
    xt = xt_ref[...]
    h = jnp.dot(w1_ref[...], xt.astype(jnp.bfloat16),
                preferred_element_type=jnp.float32)
    h = jnp.maximum(h + b1_ref[:, 0:1], 0.0)
    y = jnp.dot(w2_ref[...], h.astype(jnp.bfloat16),
                preferred_element_type=jnp.float32)
    o_ref[...] = (y + b2_ref[:, 0:1]).astype(o_ref.dtype)


@jax.jit
def _forward(x, w1_blk, b1_blk, w2_blk, b2_blk):
    B, f = x.shape
    xt = x.T

    w1 = w1_blk[:FEATS, :FEATS].T.astype(jnp.bfloat16)
    w2 = w2_blk[:FEATS, :FEATS].T.astype(jnp.bfloat16)
    b1c = jnp.tile(b1_blk[:1, :FEATS].T, (1, LANES))
    b2c = jnp.tile(b2_blk[:1, :FEATS].T, (1, LANES))

    cb = min(COL_BLOCK, B)
    grid = (pl.cdiv(B, cb),)

    yt = pl.pallas_call(
        _mlp_t_body,
        out_shape=jax.ShapeDtypeStruct((f, B), x.dtype),
        grid=grid,
        in_specs=[
            pl.BlockSpec((FEATS, cb), lambda i: (0, i)),
            pl.BlockSpec((FEATS, FEATS), lambda i: (0, 0)),
            pl.BlockSpec((FEATS, LANES), lambda i: (0, 0)),
            pl.BlockSpec((FEATS, FEATS), lambda i: (0, 0)),
            pl.BlockSpec((FEATS, LANES), lambda i: (0, 0)),
        ],
        out_specs=pl.BlockSpec((FEATS, cb), lambda i: (0, i)),
        compiler_params=pltpu.CompilerParams(
            dimension_semantics=("parallel",),
            vmem_limit_bytes=64 << 20,
        ),
    )(xt, w1, b1c, w2, b2c)

    return yt.T


def kernel(x, w1_blk, b1_blk, w2_blk, b2_blk):
    return _forward(x, w1_blk, b1_blk, w2_blk, b2_blk)


# speedup vs baseline: 1.2205x; 1.2205x over previous
import functools

import jax
import jax.numpy as jnp
from jax.experimental import pallas as pl
from jax.experimental.pallas import tpu as pltpu

FEATS = 16
LANES = 128
COL_BLOCK = 163840


def _mlp_t_body(xt_ref, w1_ref, b1_ref, w2_ref, b2_ref, o_ref):
    xt = xt_ref[...]
    h = jnp.dot(w1_ref[...], xt.astype(jnp.bfloat16),
                preferred_element_type=jnp.float32)
    h = jnp.maximum(h + b1_ref[:, 0:1], 0.0)
    y = jnp.dot(w2_ref[...], h.astype(jnp.bfloat16),
                preferred_element_type=jnp.float32)
    o_ref[...] = (y + b2_ref[:, 0:1]).astype(o_ref.dtype)


@jax.jit
def _forward(x, w1_blk, b1_blk, w2_blk, b2_blk):
    B, f = x.shape
    xt = x.T

    w1 = w1_blk[:FEATS, :FEATS].T.astype(jnp.bfloat16)
    w2 = w2_blk[:FEATS, :FEATS].T.astype(jnp.bfloat16)
    b1c = jnp.tile(b1_blk[:1, :FEATS].T, (1, LANES))
    b2c = jnp.tile(b2_blk[:1, :FEATS].T, (1, LANES))

    cb = min(COL_BLOCK, B)
    grid = (pl.cdiv(B, cb),)

    yt = pl.pallas_call(
        _mlp_t_body,
        out_shape=jax.ShapeDtypeStruct((f, B), x.dtype),
        grid=grid,
        in_specs=[
            pl.BlockSpec((FEATS, cb), lambda i: (0, i)),
            pl.BlockSpec((FEATS, FEATS), lambda i: (0, 0)),
            pl.BlockSpec((FEATS, LANES), lambda i: (0, 0)),
            pl.BlockSpec((FEATS, FEATS), lambda i: (0, 0)),
            pl.BlockSpec((FEATS, LANES), lambda i: (0, 0)),
        ],
        out_specs=pl.BlockSpec((FEATS, cb), lambda i: (0, i)),
        compiler_params=pltpu.CompilerParams(
            dimension_semantics=("parallel",),
            vmem_limit_bytes=64 << 20,
        ),
    )(xt, w1, b1c, w2, b2c)

    return yt.T


def kernel(x, w1_blk, b1_blk, w2_blk, b2_blk):
    return _forward(x, w1_blk, b1_blk, w2_blk, b2_blk)


# speedup vs baseline: 1.2252x; 1.0038x over previous
import functools

import jax
import jax.numpy as jnp
from jax.experimental import pallas as pl
from jax.experimental.pallas import tpu as pltpu

FEATS = 16
LANES = 128
COL_BLOCK = 196608


def _mlp_t_body(xt_ref, w1_ref, b1_ref, w2_ref, b2_ref, o_ref):
    xt = xt_ref[...]
    h = jnp.dot(w1_ref[...], xt.astype(jnp.bfloat16),
                preferred_element_type=jnp.float32)
    h = jnp.maximum(h + b1_ref[:, 0:1], 0.0)
    y = jnp.dot(w2_ref[...], h.astype(jnp.bfloat16),
                preferred_element_type=jnp.float32)
    o_ref[...] = (y + b2_ref[:, 0:1]).astype(o_ref.dtype)


@jax.jit
def _forward(x, w1_blk, b1_blk, w2_blk, b2_blk):
    B, f = x.shape
    xt = x.T

    w1 = w1_blk[:FEATS, :FEATS].T.astype(jnp.bfloat16)
    w2 = w2_blk[:FEATS, :FEATS].T.astype(jnp.bfloat16)
    b1c = jnp.tile(b1_blk[:1, :FEATS].T, (1, LANES))
    b2c = jnp.tile(b2_blk[:1, :FEATS].T, (1, LANES))

    cb = min(COL_BLOCK, B)
    grid = (pl.cdiv(B, cb),)

    yt = pl.pallas_call(
        _mlp_t_body,
        out_shape=jax.ShapeDtypeStruct((f, B), x.dtype),
        grid=grid,
        in_specs=[
            pl.BlockSpec((FEATS, cb), lambda i: (0, i)),
            pl.BlockSpec((FEATS, FEATS), lambda i: (0, 0)),
            pl.BlockSpec((FEATS, LANES), lambda i: (0, 0)),
            pl.BlockSpec((FEATS, FEATS), lambda i: (0, 0)),
            pl.BlockSpec((FEATS, LANES), lambda i: (0, 0)),
        ],
        out_specs=pl.BlockSpec((FEATS, cb), lambda i: (0, i)),
        compiler_params=pltpu.CompilerParams(
            dimension_semantics=("parallel",),
            vmem_limit_bytes=64 << 20,
        ),
    )(xt, w1, b1c, w2, b2c)

    return yt.T


def kernel(x, w1_blk, b1_blk, w2_blk, b2_blk):
    return _forward(x, w1_blk, b1_blk, w2_blk, b2_blk)


# speedup vs baseline: 1.2289x; 1.0030x over previous
import functools

import jax
import jax.numpy as jnp
from jax.experimental import pallas as pl
from jax.experimental.pallas import tpu as pltpu

FEATS = 16
LANES = 128
COL_BLOCK = 229376


def _mlp_t_body(xt_ref, w1_ref, b1_ref, w2_ref, b2_ref, o_ref):
    xt = xt_ref[...]
    h = jnp.dot(w1_ref[...], xt.astype(jnp.bfloat16),
                preferred_element_type=jnp.float32)
    h = jnp.maximum(h + b1_ref[:, 0:1], 0.0)
    y = jnp.dot(w2_ref[...], h.astype(jnp.bfloat16),
                preferred_element_type=jnp.float32)
    o_ref[...] = (y + b2_ref[:, 0:1]).astype(o_ref.dtype)


@jax.jit
def _forward(x, w1_blk, b1_blk, w2_blk, b2_blk):
    B, f = x.shape
    xt = x.T

    w1 = w1_blk[:FEATS, :FEATS].T.astype(jnp.bfloat16)
    w2 = w2_blk[:FEATS, :FEATS].T.astype(jnp.bfloat16)
    b1c = jnp.tile(b1_blk[:1, :FEATS].T, (1, LANES))
    b2c = jnp.tile(b2_blk[:1, :FEATS].T, (1, LANES))

    cb = min(COL_BLOCK, B)
    grid = (pl.cdiv(B, cb),)

    yt = pl.pallas_call(
        _mlp_t_body,
        out_shape=jax.ShapeDtypeStruct((f, B), x.dtype),
        grid=grid,
        in_specs=[
            pl.BlockSpec((FEATS, cb), lambda i: (0, i)),
            pl.BlockSpec((FEATS, FEATS), lambda i: (0, 0)),
            pl.BlockSpec((FEATS, LANES), lambda i: (0, 0)),
            pl.BlockSpec((FEATS, FEATS), lambda i: (0, 0)),
            pl.BlockSpec((FEATS, LANES), lambda i: (0, 0)),
        ],
        out_specs=pl.BlockSpec((FEATS, cb), lambda i: (0, i)),
        compiler_params=pltpu.CompilerParams(
            dimension_semantics=("parallel",),
            vmem_limit_bytes=64 << 20,
        ),
    )(xt, w1, b1c, w2, b2c)

    return yt.T


def kernel(x, w1_blk, b1_blk, w2_blk, b2_blk):
    return _forward(x, w1_blk, b1_blk, w2_blk, b2_blk)


# speedup vs baseline: 1.2303x; 1.0012x over previous
import jax
import jax.numpy as jnp
from jax.experimental import pallas as pl
from jax.experimental.pallas import tpu as pltpu

FEATS = 16
LANES = 128
COL_BLOCK = 229376


def _mlp_t_body(xt_ref, w1_ref, b1_ref, w2_ref, b2_ref, o_ref):
    xt = xt_ref[...]
    h = jnp.dot(w1_ref[...], xt.astype(jnp.bfloat16),
                preferred_element_type=jnp.float32)
    h = jnp.maximum(h + b1_ref[:, 0:1], 0.0)
    y = jnp.dot(w2_ref[...], h.astype(jnp.bfloat16),
                preferred_element_type=jnp.float32)
    o_ref[...] = (y + b2_ref[:, 0:1]).astype(o_ref.dtype)


@jax.jit
def _forward(x, w1_blk, b1_blk, w2_blk, b2_blk):
    B, f = x.shape
    xt = x.T

    w1 = w1_blk[:FEATS, :FEATS].T.astype(jnp.bfloat16)
    w2 = w2_blk[:FEATS, :FEATS].T.astype(jnp.bfloat16)
    b1c = jnp.tile(b1_blk[:1, :FEATS].T, (1, LANES))
    b2c = jnp.tile(b2_blk[:1, :FEATS].T, (1, LANES))

    cb = min(COL_BLOCK, B)
    grid = (pl.cdiv(B, cb),)

    yt = pl.pallas_call(
        _mlp_t_body,
        out_shape=jax.ShapeDtypeStruct((f, B), x.dtype),
        grid=grid,
        in_specs=[
            pl.BlockSpec((FEATS, cb), lambda i: (0, i)),
            pl.BlockSpec((FEATS, FEATS), lambda i: (0, 0)),
            pl.BlockSpec((FEATS, LANES), lambda i: (0, 0)),
            pl.BlockSpec((FEATS, FEATS), lambda i: (0, 0)),
            pl.BlockSpec((FEATS, LANES), lambda i: (0, 0)),
        ],
        out_specs=pl.BlockSpec((FEATS, cb), lambda i: (0, i)),
        compiler_params=pltpu.CompilerParams(
            dimension_semantics=("parallel",),
            vmem_limit_bytes=64 << 20,
        ),
    )(xt, w1, b1c, w2, b2c)

    return yt.T


def kernel(x, w1_blk, b1_blk, w2_blk, b2_blk):
    return _forward(x, w1_blk, b1_blk, w2_blk, b2_blk)
